# baseline (speedup 1.0000x reference)
import functools

import jax
import jax.numpy as jnp
from jax import lax
from jax.experimental import pallas as pl
from jax.experimental.pallas import tpu as pltpu
from jax.experimental.pallas import tpu_sc as plsc

_NC = 2
_NS = 16
_C = 80
_AUG = 32


def _sc_segment_sums(feat2, eidx3, ef, n):
  dh = feat2.shape[-1]
  e_total = eidx3.shape[-1]
  de = ef.shape[-1]
  ept = e_total // _NS
  nch = ept // _C
  npad = ((n + _C * _NS - 1) // (_C * _NS)) * (_C * _NS)
  rpt = npad // _NS
  nef = nch // _NC

  mesh = plsc.VectorSubcoreMesh(core_axis_name="c", subcore_axis_name="s")

  @functools.partial(
      pl.kernel,
      out_type=[
          jax.ShapeDtypeStruct((_NC, npad, dh), jnp.float32),
          jax.ShapeDtypeStruct((_NC, npad, _AUG), jnp.float32),
      ],
      mesh=mesh,
      scratch_types=[
          pltpu.VMEM((ept,), jnp.int32),
          pltpu.VMEM((ept,), jnp.int32),
          pltpu.VMEM((_C, dh), jnp.float32),
          pltpu.VMEM((_C, dh), jnp.float32),
          pltpu.VMEM((_C, _AUG), jnp.float32),
          pltpu.VMEM((_C, _AUG), jnp.float32),
          pltpu.VMEM((_C, dh), jnp.float32),
          pltpu.VMEM((_C, _AUG), jnp.float32),
          pltpu.VMEM_SHARED((npad, dh), jnp.float32),
          pltpu.VMEM_SHARED((npad, _AUG), jnp.float32),
          pltpu.SemaphoreType.DMA,
          pltpu.SemaphoreType.DMA,
          pltpu.SemaphoreType.DMA,
          pltpu.SemaphoreType.DMA,
      ],
      compiler_params=pltpu.CompilerParams(use_tc_tiling_on_sc=False),
  )
  def k(eidx_h, ef_h, feat2_h, psum_h, paug_h,
        sidx, didx, r0b, r1b, a0b, a1b, zb, zab, acc, eacc,
        g0, g1, e0, e1):
    cid = lax.axis_index("c")
    sid = lax.axis_index("s")
    base = sid * ept

    pltpu.sync_copy(eidx_h.at[cid, pl.ds(base, ept)], sidx)
    pltpu.sync_copy(eidx_h.at[2, pl.ds(base, ept)], didx)

    def gsrc(ci):
      return feat2_h.at[sidx.at[pl.ds(ci * _C, _C)]]

    def dst_at(ci):
      return didx.at[pl.ds(ci * _C, _C)]

    pltpu.async_copy(gsrc(0), r0b, g0)
    pltpu.async_copy(gsrc(1), r1b, g1)

    zeros16 = jnp.zeros((16,), jnp.float32)
    onecol = jnp.where(lax.iota(jnp.int32, 16) == 0,
                       jnp.float32(1.0), jnp.float32(0.0))

    def zrow(i, carry):
      for j in range(dh // 16):
        zb[i, pl.ds(j * 16, 16)] = zeros16
      for j in range(_AUG // 16):
        zab[i, pl.ds(j * 16, 16)] = zeros16
      a0b[i, pl.ds(de, 16)] = onecol
      a1b[i, pl.ds(de, 16)] = onecol
      return carry
    lax.fori_loop(0, _C, zrow, 0)

    rr0 = sid * rpt
    for kk in range(rpt // _C):
      pltpu.sync_copy(zb, acc.at[pl.ds(rr0 + kk * _C, _C)])
      pltpu.sync_copy(zab, eacc.at[pl.ds(rr0 + kk * _C, _C)])
    plsc.subcore_barrier()

    eoff = cid * nef

    def ef_at(ei):
      return ef_h.at[pl.ds(base + ei * _C, _C)]

    pltpu.async_copy(ef_at(eoff), a0b.at[:, pl.ds(0, de)], e0)
    pltpu.async_copy(ef_at(eoff + 1), a1b.at[:, pl.ds(0, de)], e1)

    def gwait(buf, sem, ci):
      pltpu.make_async_copy(gsrc(ci), buf, sem).wait()

    def ewait(buf, sem, ei):
      pltpu.make_async_copy(ef_at(ei), buf.at[:, pl.ds(0, de)], sem).wait()

    def body(g, carry):
      for (buf, sem, off) in ((r0b, g0, 0), (r1b, g1, 1),
                              (r0b, g0, 2), (r1b, g1, 3)):
        ci = 4 * g + off
        gwait(buf, sem, ci)
        pltpu.sync_copy(buf, acc.at[dst_at(ci)], add=True)
        pltpu.async_copy(gsrc(ci + 2), buf, sem)
      for (buf, sem, off) in ((a0b, e0, 0), (a1b, e1, 1)):
        ei = eoff + 2 * g + off
        ewait(buf, sem, ei)
        pltpu.sync_copy(buf, eacc.at[dst_at(ei)], add=True)
        nxt = 2 * g + off + 2

        @pl.when(nxt < nef)
        def _():
          pltpu.async_copy(ef_at(eoff + nxt), buf.at[:, pl.ds(0, de)], sem)
      return carry

    niter = (nch - 2) // 4
    lax.fori_loop(0, niter, body, 0)

    gwait(r0b, g0, nch - 2)
    pltpu.sync_copy(r0b, acc.at[dst_at(nch - 2)], add=True)
    gwait(r1b, g1, nch - 1)
    pltpu.sync_copy(r1b, acc.at[dst_at(nch - 1)], add=True)
    if nef % 2:
      ewait(a0b, e0, eoff + nef - 1)
      pltpu.sync_copy(a0b, eacc.at[dst_at(eoff + nef - 1)], add=True)

    plsc.subcore_barrier()
    pltpu.sync_copy(acc.at[pl.ds(rr0, rpt)],
                    psum_h.at[cid, pl.ds(rr0, rpt)])
    pltpu.sync_copy(eacc.at[pl.ds(rr0, rpt)],
                    paug_h.at[cid, pl.ds(rr0, rpt)])

  return k(eidx3, ef, feat2)


def _combine(feat, psum, paug, w_self, w_neigh, bias2d):
  n, d = feat.shape
  dh = psum.shape[-1]
  dk = w_neigh.shape[0]
  de = dk - d
  blk = 1000
  grid = (n // blk,)

  def body(f_ref, ps_ref, pa_ref, ws_ref, wn_ref, b_ref, o_ref):
    a = pa_ref[0] + pa_ref[1]
    e = a[:, 0:de]
    dg = a[:, de:de + 1]
    scale = 1.0 / jnp.maximum(dg, 1.0)
    wn = wn_ref[...]
    h = (jnp.dot(ps_ref[0], wn[0:dh], preferred_element_type=jnp.float32)
         + jnp.dot(ps_ref[1], wn[dh:2 * dh],
                   preferred_element_type=jnp.float32)
         + jnp.dot(e, wn[2 * dh:dk], preferred_element_type=jnp.float32))
    o_ref[...] = (jnp.dot(f_ref[...], ws_ref[...],
                          preferred_element_type=jnp.float32)
                  + h * scale + b_ref[...])

  return pl.pallas_call(
      body,
      grid=grid,
      in_specs=[
          pl.BlockSpec((blk, d), lambda i: (i, 0)),
          pl.BlockSpec((_NC, blk, dh), lambda i: (0, i, 0)),
          pl.BlockSpec((_NC, blk, _AUG), lambda i: (0, i, 0)),
          pl.BlockSpec((d, d), lambda i: (0, 0)),
          pl.BlockSpec((dk, d), lambda i: (0, 0)),
          pl.BlockSpec((1, d), lambda i: (0, 0)),
      ],
      out_specs=pl.BlockSpec((blk, d), lambda i: (i, 0)),
      out_shape=jax.ShapeDtypeStruct((n, d), jnp.float32),
  )(feat, psum, paug, w_self, w_neigh, bias2d)


def kernel(feat, edge_index, edge_feats, W_self, W_neigh, bias):
  n, d = feat.shape
  dh = d // _NC
  src = edge_index[0]
  dst = edge_index[1]
  feat2 = jnp.concatenate([feat[:, :dh], feat[:, dh:]], axis=0)
  eidx3 = jnp.stack([src, src + n, dst])
  psum, paug = _sc_segment_sums(feat2, eidx3, edge_feats, n)
  return _combine(feat, psum, paug, W_self, W_neigh, bias.reshape(1, d))


# speedup vs baseline: 1.2090x; 1.2090x over previous
import functools

import jax
import jax.numpy as jnp
from jax import lax
from jax.experimental import pallas as pl
from jax.experimental.pallas import tpu as pltpu
from jax.experimental.pallas import tpu_sc as plsc

_NC = 2
_NS = 16
_C = 80


def _npad_for(n):
  return ((n + _C * _NS - 1) // (_C * _NS)) * (_C * _NS)


def _mesh():
  return plsc.VectorSubcoreMesh(core_axis_name="c", subcore_axis_name="s")


_LINEAR = pltpu.CompilerParams(use_tc_tiling_on_sc=False)


def _sc_feat_deg_sums(feat2, eidx3, n, de):
  dh = feat2.shape[-1]
  e_total = eidx3.shape[-1]
  ept = e_total // _NS
  nch = ept // _C
  npad = _npad_for(n)
  rpt = npad // _NS
  nef = nch // _NC

  @functools.partial(
      pl.kernel,
      out_type=[
          jax.ShapeDtypeStruct((_NC, npad, dh), jnp.float32),
          jax.ShapeDtypeStruct((_NC, npad, de), jnp.float32),
      ],
      mesh=_mesh(),
      scratch_types=[
          pltpu.VMEM((ept,), jnp.int32),
          pltpu.VMEM((ept,), jnp.int32),
          pltpu.VMEM((_C, dh), jnp.float32),
          pltpu.VMEM((_C, dh), jnp.float32),
          pltpu.VMEM((_C, dh), jnp.float32),
          pltpu.VMEM((_C, de), jnp.float32),
          pltpu.VMEM_SHARED((npad, dh), jnp.float32),
          pltpu.VMEM_SHARED((npad, de), jnp.float32),
          pltpu.SemaphoreType.DMA,
          pltpu.SemaphoreType.DMA,
      ],
      compiler_params=_LINEAR,
  )
  def k(eidx_h, feat2_h, psum_h, pdeg_h,
        sidx, didx, r0b, r1b, zb, onesb, acc, dacc, g0, g1):
    cid = lax.axis_index("c")
    sid = lax.axis_index("s")
    base = sid * ept

    pltpu.sync_copy(eidx_h.at[cid, pl.ds(base, ept)], sidx)
    pltpu.sync_copy(eidx_h.at[2, pl.ds(base, ept)], didx)

    def gsrc(ci):
      return feat2_h.at[sidx.at[pl.ds(ci * _C, _C)]]

    def dst_at(ci):
      return didx.at[pl.ds(ci * _C, _C)]

    pltpu.async_copy(gsrc(0), r0b, g0)
    pltpu.async_copy(gsrc(1), r1b, g1)

    zeros16 = jnp.zeros((16,), jnp.float32)
    ones16 = jnp.ones((16,), jnp.float32)

    def zrow(i, carry):
      for j in range(dh // 16):
        zb[i, pl.ds(j * 16, 16)] = zeros16
      for j in range(de // 16):
        onesb[i, pl.ds(j * 16, 16)] = ones16
      return carry
    lax.fori_loop(0, _C, zrow, 0)

    rr0 = sid * rpt
    for kk in range(rpt // _C):
      pltpu.sync_copy(zb, acc.at[pl.ds(rr0 + kk * _C, _C)])
      pltpu.sync_copy(zb.at[:, pl.ds(0, de)],
                      dacc.at[pl.ds(rr0 + kk * _C, _C)])
    plsc.subcore_barrier()

    def gwait(buf, sem, ci):
      pltpu.make_async_copy(gsrc(ci), buf, sem).wait()

    eoff = cid * nef

    def body(g, carry):
      for (buf, sem, off) in ((r0b, g0, 0), (r1b, g1, 1),
                              (r0b, g0, 2), (r1b, g1, 3)):
        ci = 4 * g + off
        gwait(buf, sem, ci)
        pltpu.sync_copy(buf, acc.at[dst_at(ci)], add=True)
        pltpu.async_copy(gsrc(ci + 2), buf, sem)
      pltpu.sync_copy(onesb, dacc.at[dst_at(eoff + 2 * g)], add=True)
      pltpu.sync_copy(onesb, dacc.at[dst_at(eoff + 2 * g + 1)], add=True)
      return carry

    niter = (nch - 2) // 4
    lax.fori_loop(0, niter, body, 0)

    gwait(r0b, g0, nch - 2)
    pltpu.sync_copy(r0b, acc.at[dst_at(nch - 2)], add=True)
    gwait(r1b, g1, nch - 1)
    pltpu.sync_copy(r1b, acc.at[dst_at(nch - 1)], add=True)
    if nef % 2:
      pltpu.sync_copy(onesb, dacc.at[dst_at(eoff + nef - 1)], add=True)

    plsc.subcore_barrier()
    pltpu.sync_copy(acc.at[pl.ds(rr0, rpt)],
                    psum_h.at[cid, pl.ds(rr0, rpt)])
    pltpu.sync_copy(dacc.at[pl.ds(rr0, rpt)],
                    pdeg_h.at[cid, pl.ds(rr0, rpt)])

  return k(eidx3, feat2)


def _sc_ef_sums(ef, eidx3, n):
  e_total = eidx3.shape[-1]
  de = ef.shape[-1]
  ept = e_total // _NS
  nch = ept // _C
  npad = _npad_for(n)
  rpt = npad // _NS
  nef = nch // _NC

  @functools.partial(
      pl.kernel,
      out_type=jax.ShapeDtypeStruct((_NC, npad, de), jnp.float32),
      mesh=_mesh(),
      scratch_types=[
          pltpu.VMEM((ept,), jnp.int32),
          pltpu.VMEM((_C, de), jnp.float32),
          pltpu.VMEM((_C, de), jnp.float32),
          pltpu.VMEM((_C, de), jnp.float32),
          pltpu.VMEM_SHARED((npad, de), jnp.float32),
          pltpu.SemaphoreType.DMA,
          pltpu.SemaphoreType.DMA,
      ],
      compiler_params=_LINEAR,
  )
  def k(eidx_h, ef_h, pef_h, didx, e0b, e1b, zb, eacc, e0, e1):
    cid = lax.axis_index("c")
    sid = lax.axis_index("s")
    base = sid * ept
    eoff = cid * nef

    pltpu.sync_copy(eidx_h.at[2, pl.ds(base, ept)], didx)

    def ef_at(ei):
      return ef_h.at[pl.ds(base + ei * _C, _C)]

    def dst_at(ci):
      return didx.at[pl.ds(ci * _C, _C)]

    pltpu.async_copy(ef_at(eoff), e0b, e0)
    pltpu.async_copy(ef_at(eoff + 1), e1b, e1)

    zeros16 = jnp.zeros((16,), jnp.float32)

    def zrow(i, carry):
      for j in range(de // 16):
        zb[i, pl.ds(j * 16, 16)] = zeros16
      return carry
    lax.fori_loop(0, _C, zrow, 0)

    rr0 = sid * rpt
    for kk in range(rpt // _C):
      pltpu.sync_copy(zb, eacc.at[pl.ds(rr0 + kk * _C, _C)])
    plsc.subcore_barrier()

    def ewait(buf, sem, ei):
      pltpu.make_async_copy(ef_at(ei), buf, sem).wait()

    def body(g, carry):
      for (buf, sem, off) in ((e0b, e0, 0), (e1b, e1, 1)):
        ei = eoff + 2 * g + off
        ewait(buf, sem, ei)
        pltpu.sync_copy(buf, eacc.at[dst_at(ei)], add=True)
        nxt = 2 * g + off + 2

        @pl.when(nxt < nef)
        def _():
          pltpu.async_copy(ef_at(eoff + nxt), buf, sem)
      return carry

    lax.fori_loop(0, nef // 2, body, 0)

    if nef % 2:
      ewait(e0b, e0, eoff + nef - 1)
      pltpu.sync_copy(e0b, eacc.at[dst_at(eoff + nef - 1)], add=True)

    plsc.subcore_barrier()
    pltpu.sync_copy(eacc.at[pl.ds(rr0, rpt)],
                    pef_h.at[cid, pl.ds(rr0, rpt)])

  return k(eidx3, ef)


def _self_term(feat, w_self, bias2d):
  n, d = feat.shape
  blk = 1000
  grid = (n // blk,)

  def body(f_ref, ws_ref, b_ref, o_ref):
    o_ref[...] = (jnp.dot(f_ref[...], ws_ref[...],
                          preferred_element_type=jnp.float32) + b_ref[...])

  return pl.pallas_call(
      body,
      grid=grid,
      in_specs=[
          pl.BlockSpec((blk, d), lambda i: (i, 0)),
          pl.BlockSpec((d, d), lambda i: (0, 0)),
          pl.BlockSpec((1, d), lambda i: (0, 0)),
      ],
      out_specs=pl.BlockSpec((blk, d), lambda i: (i, 0)),
      out_shape=jax.ShapeDtypeStruct((n, d), jnp.float32),
  )(feat, w_self, bias2d)


def _combine(selfterm, psum, pdeg, pef, w_neigh):
  n, d = selfterm.shape
  dh = psum.shape[-1]
  de = pef.shape[-1]
  dk = w_neigh.shape[0]
  blk = 1000
  grid = (n // blk,)

  def body(st_ref, ps_ref, pd_ref, pe_ref, wn_ref, o_ref):
    dg = pd_ref[0, :, 0:1] + pd_ref[1, :, 0:1]
    e = pe_ref[0] + pe_ref[1]
    scale = 1.0 / jnp.maximum(dg, 1.0)
    wn = wn_ref[...]
    h = (jnp.dot(ps_ref[0], wn[0:dh], preferred_element_type=jnp.float32)
         + jnp.dot(ps_ref[1], wn[dh:2 * dh],
                   preferred_element_type=jnp.float32)
         + jnp.dot(e, wn[2 * dh:dk], preferred_element_type=jnp.float32))
    o_ref[...] = st_ref[...] + h * scale

  return pl.pallas_call(
      body,
      grid=grid,
      in_specs=[
          pl.BlockSpec((blk, d), lambda i: (i, 0)),
          pl.BlockSpec((_NC, blk, dh), lambda i: (0, i, 0)),
          pl.BlockSpec((_NC, blk, de), lambda i: (0, i, 0)),
          pl.BlockSpec((_NC, blk, de), lambda i: (0, i, 0)),
          pl.BlockSpec((dk, d), lambda i: (0, 0)),
      ],
      out_specs=pl.BlockSpec((blk, d), lambda i: (i, 0)),
      out_shape=jax.ShapeDtypeStruct((n, d), jnp.float32),
  )(selfterm, psum, pdeg, pef, w_neigh)


def kernel(feat, edge_index, edge_feats, W_self, W_neigh, bias):
  n, d = feat.shape
  de = edge_feats.shape[1]
  dh = d // _NC
  src = edge_index[0]
  dst = edge_index[1]
  feat2 = jnp.concatenate([feat[:, :dh], feat[:, dh:]], axis=0)
  eidx3 = jnp.stack([src, src + n, dst])
  psum, pdeg = _sc_feat_deg_sums(feat2, eidx3, n, de)
  pef = _sc_ef_sums(edge_feats, eidx3, n)
  st = _self_term(feat, W_self, bias.reshape(1, d))
  return _combine(st, psum, pdeg, pef, W_neigh)


# speedup vs baseline: 1.2790x; 1.0579x over previous
import functools

import jax
import jax.numpy as jnp
from jax import lax
from jax.experimental import pallas as pl
from jax.experimental.pallas import tpu as pltpu
from jax.experimental.pallas import tpu_sc as plsc

_NC = 2
_NS = 16
_C = 80


def _npad_for(n):
  return ((n + _C * _NS - 1) // (_C * _NS)) * (_C * _NS)


def _mesh():
  return plsc.VectorSubcoreMesh(core_axis_name="c", subcore_axis_name="s")


_LINEAR = pltpu.CompilerParams(use_tc_tiling_on_sc=False)


def _sc_feat_deg_sums(feat2, eidx3, n, de):
  dh = feat2.shape[-1]
  e_total = eidx3.shape[-1]
  ept = e_total // _NS
  nch = ept // _C
  npad = _npad_for(n)
  rpt = npad // _NS
  nef = nch // _NC

  @functools.partial(
      pl.kernel,
      out_type=[
          jax.ShapeDtypeStruct((_NC, npad, dh), jnp.float32),
          jax.ShapeDtypeStruct((_NC, npad, de), jnp.float32),
      ],
      mesh=_mesh(),
      scratch_types=[
          pltpu.VMEM((ept,), jnp.int32),
          pltpu.VMEM((ept,), jnp.int32),
          pltpu.VMEM((_C, dh), jnp.float32),
          pltpu.VMEM((_C, dh), jnp.float32),
          pltpu.VMEM((_C, dh), jnp.float32),
          pltpu.VMEM((_C, dh), jnp.float32),
          pltpu.VMEM((_C, dh), jnp.float32),
          pltpu.VMEM((_C, de), jnp.float32),
          pltpu.VMEM_SHARED((npad, dh), jnp.float32),
          pltpu.VMEM_SHARED((npad, de), jnp.float32),
          pltpu.SemaphoreType.DMA,
          pltpu.SemaphoreType.DMA,
          pltpu.SemaphoreType.DMA,
          pltpu.SemaphoreType.DMA,
      ],
      compiler_params=_LINEAR,
  )
  def k(eidx_h, feat2_h, psum_h, pdeg_h,
        sidx, didx, r0b, r1b, r2b, r3b, zb, onesb, acc, dacc,
        g0, g1, g2, g3):
    cid = lax.axis_index("c")
    sid = lax.axis_index("s")
    base = sid * ept

    pltpu.sync_copy(eidx_h.at[cid, pl.ds(base, ept)], sidx)
    pltpu.sync_copy(eidx_h.at[2, pl.ds(base, ept)], didx)

    def gsrc(ci):
      return feat2_h.at[sidx.at[pl.ds(ci * _C, _C)]]

    def dst_at(ci):
      return didx.at[pl.ds(ci * _C, _C)]

    ring = ((r0b, g0), (r1b, g1), (r2b, g2), (r3b, g3))

    pltpu.async_copy(gsrc(0), r0b, g0)
    pltpu.async_copy(gsrc(1), r1b, g1)

    zeros16 = jnp.zeros((16,), jnp.float32)
    ones16 = jnp.ones((16,), jnp.float32)

    def zrow(i, carry):
      for j in range(dh // 16):
        zb[i, pl.ds(j * 16, 16)] = zeros16
      for j in range(de // 16):
        onesb[i, pl.ds(j * 16, 16)] = ones16
      return carry
    lax.fori_loop(0, _C, zrow, 0)

    rr0 = sid * rpt
    for kk in range(rpt // _C):
      pltpu.sync_copy(zb, acc.at[pl.ds(rr0 + kk * _C, _C)])
      pltpu.sync_copy(zb.at[:, pl.ds(0, de)],
                      dacc.at[pl.ds(rr0 + kk * _C, _C)])
    plsc.subcore_barrier()

    def gwait(buf, sem, ci):
      pltpu.make_async_copy(gsrc(ci), buf, sem).wait()

    def sissue(buf, sem, ci):
      pltpu.make_async_copy(buf, acc.at[dst_at(ci)], sem).start(add=True)

    def swait(buf, sem, ci):
      pltpu.make_async_copy(buf, acc.at[dst_at(ci)], sem).wait()

    eoff = cid * nef

    gwait(r0b, g0, 0)
    sissue(r0b, g0, 0)
    pltpu.async_copy(gsrc(2), r2b, g2)
    gwait(r1b, g1, 1)
    sissue(r1b, g1, 1)
    pltpu.async_copy(gsrc(3), r3b, g3)

    def body(g, carry):
      for off in range(4):
        ci = 4 * g + 2 + off
        buf, sem = ring[(2 + off) % 4]
        gwait(buf, sem, ci)
        sissue(buf, sem, ci)
        nbuf, nsem = ring[(4 + off) % 4]
        swait(nbuf, nsem, ci - 2)

        @pl.when(ci + 2 < nch)
        def _():
          pltpu.async_copy(gsrc(ci + 2), nbuf, nsem)
      pltpu.sync_copy(onesb, dacc.at[dst_at(eoff + 2 * g)], add=True)
      pltpu.sync_copy(onesb, dacc.at[dst_at(eoff + 2 * g + 1)], add=True)
      return carry

    niter = (nch - 2) // 4
    lax.fori_loop(0, niter, body, 0)

    swait(ring[0][0], ring[0][1], nch - 2)
    swait(ring[1][0], ring[1][1], nch - 1)
    for ci in range(2 * niter, nef):
      pltpu.sync_copy(onesb, dacc.at[dst_at(eoff + ci)], add=True)

    plsc.subcore_barrier()
    pltpu.sync_copy(acc.at[pl.ds(rr0, rpt)],
                    psum_h.at[cid, pl.ds(rr0, rpt)])
    pltpu.sync_copy(dacc.at[pl.ds(rr0, rpt)],
                    pdeg_h.at[cid, pl.ds(rr0, rpt)])

  return k(eidx3, feat2)


def _sc_ef_sums(ef, eidx3, n):
  e_total = eidx3.shape[-1]
  de = ef.shape[-1]
  ept = e_total // _NS
  nch = ept // _C
  npad = _npad_for(n)
  rpt = npad // _NS
  nef = nch // _NC

  @functools.partial(
      pl.kernel,
      out_type=jax.ShapeDtypeStruct((_NC, npad, de), jnp.float32),
      mesh=_mesh(),
      scratch_types=[
          pltpu.VMEM((ept,), jnp.int32),
          pltpu.VMEM((_C, de), jnp.float32),
          pltpu.VMEM((_C, de), jnp.float32),
          pltpu.VMEM((_C, de), jnp.float32),
          pltpu.VMEM_SHARED((npad, de), jnp.float32),
          pltpu.SemaphoreType.DMA,
          pltpu.SemaphoreType.DMA,
      ],
      compiler_params=_LINEAR,
  )
  def k(eidx_h, ef_h, pef_h, didx, e0b, e1b, zb, eacc, e0, e1):
    cid = lax.axis_index("c")
    sid = lax.axis_index("s")
    base = sid * ept
    eoff = cid * nef

    pltpu.sync_copy(eidx_h.at[2, pl.ds(base, ept)], didx)

    def ef_at(ei):
      return ef_h.at[pl.ds(base + ei * _C, _C)]

    def dst_at(ci):
      return didx.at[pl.ds(ci * _C, _C)]

    pltpu.async_copy(ef_at(eoff), e0b, e0)
    pltpu.async_copy(ef_at(eoff + 1), e1b, e1)

    zeros16 = jnp.zeros((16,), jnp.float32)

    def zrow(i, carry):
      for j in range(de // 16):
        zb[i, pl.ds(j * 16, 16)] = zeros16
      return carry
    lax.fori_loop(0, _C, zrow, 0)

    rr0 = sid * rpt
    for kk in range(rpt // _C):
      pltpu.sync_copy(zb, eacc.at[pl.ds(rr0 + kk * _C, _C)])
    plsc.subcore_barrier()

    def ewait(buf, sem, ei):
      pltpu.make_async_copy(ef_at(ei), buf, sem).wait()

    def body(g, carry):
      for (buf, sem, off) in ((e0b, e0, 0), (e1b, e1, 1)):
        ei = eoff + 2 * g + off
        ewait(buf, sem, ei)
        pltpu.sync_copy(buf, eacc.at[dst_at(ei)], add=True)
        nxt = 2 * g + off + 2

        @pl.when(nxt < nef)
        def _():
          pltpu.async_copy(ef_at(eoff + nxt), buf, sem)
      return carry

    lax.fori_loop(0, nef // 2, body, 0)

    if nef % 2:
      ewait(e0b, e0, eoff + nef - 1)
      pltpu.sync_copy(e0b, eacc.at[dst_at(eoff + nef - 1)], add=True)

    plsc.subcore_barrier()
    pltpu.sync_copy(eacc.at[pl.ds(rr0, rpt)],
                    pef_h.at[cid, pl.ds(rr0, rpt)])

  return k(eidx3, ef)


def _self_term(feat, w_self, bias2d):
  n, d = feat.shape
  blk = 1000
  grid = (n // blk,)

  def body(f_ref, ws_ref, b_ref, o_ref):
    o_ref[...] = (jnp.dot(f_ref[...], ws_ref[...],
                          preferred_element_type=jnp.float32) + b_ref[...])

  return pl.pallas_call(
      body,
      grid=grid,
      in_specs=[
          pl.BlockSpec((blk, d), lambda i: (i, 0)),
          pl.BlockSpec((d, d), lambda i: (0, 0)),
          pl.BlockSpec((1, d), lambda i: (0, 0)),
      ],
      out_specs=pl.BlockSpec((blk, d), lambda i: (i, 0)),
      out_shape=jax.ShapeDtypeStruct((n, d), jnp.float32),
  )(feat, w_self, bias2d)


def _combine(selfterm, psum, pdeg, pef, w_neigh):
  n, d = selfterm.shape
  dh = psum.shape[-1]
  de = pef.shape[-1]
  dk = w_neigh.shape[0]
  blk = 1000
  grid = (n // blk,)

  def body(st_ref, ps_ref, pd_ref, pe_ref, wn_ref, o_ref):
    dg = pd_ref[0, :, 0:1] + pd_ref[1, :, 0:1]
    e = pe_ref[0] + pe_ref[1]
    scale = 1.0 / jnp.maximum(dg, 1.0)
    wn = wn_ref[...]
    h = (jnp.dot(ps_ref[0], wn[0:dh], preferred_element_type=jnp.float32)
         + jnp.dot(ps_ref[1], wn[dh:2 * dh],
                   preferred_element_type=jnp.float32)
         + jnp.dot(e, wn[2 * dh:dk], preferred_element_type=jnp.float32))
    o_ref[...] = st_ref[...] + h * scale

  return pl.pallas_call(
      body,
      grid=grid,
      in_specs=[
          pl.BlockSpec((blk, d), lambda i: (i, 0)),
          pl.BlockSpec((_NC, blk, dh), lambda i: (0, i, 0)),
          pl.BlockSpec((_NC, blk, de), lambda i: (0, i, 0)),
          pl.BlockSpec((_NC, blk, de), lambda i: (0, i, 0)),
          pl.BlockSpec((dk, d), lambda i: (0, 0)),
      ],
      out_specs=pl.BlockSpec((blk, d), lambda i: (i, 0)),
      out_shape=jax.ShapeDtypeStruct((n, d), jnp.float32),
  )(selfterm, psum, pdeg, pef, w_neigh)


def kernel(feat, edge_index, edge_feats, W_self, W_neigh, bias):
  n, d = feat.shape
  de = edge_feats.shape[1]
  dh = d // _NC
  src = edge_index[0]
  dst = edge_index[1]
  feat2 = jnp.concatenate([feat[:, :dh], feat[:, dh:]], axis=0)
  eidx3 = jnp.stack([src, src + n, dst])
  psum, pdeg = _sc_feat_deg_sums(feat2, eidx3, n, de)
  pef = _sc_ef_sums(edge_feats, eidx3, n)
  st = _self_term(feat, W_self, bias.reshape(1, d))
  return _combine(st, psum, pdeg, pef, W_neigh)


# speedup vs baseline: 1.3061x; 1.0212x over previous
import functools

import jax
import jax.numpy as jnp
from jax import lax
from jax.experimental import pallas as pl
from jax.experimental.pallas import tpu as pltpu
from jax.experimental.pallas import tpu_sc as plsc

_NC = 2
_NS = 16
_C = 80


def _npad_for(n):
  return ((n + _C * _NS - 1) // (_C * _NS)) * (_C * _NS)


def _mesh():
  return plsc.VectorSubcoreMesh(core_axis_name="c", subcore_axis_name="s")


_LINEAR = pltpu.CompilerParams(use_tc_tiling_on_sc=False)


def _sc_feat_deg_sums(feat2, eidx3, n, de):
  dh = feat2.shape[-1]
  e_total = eidx3.shape[-1]
  ept = e_total // _NS
  nch = ept // _C
  npad = _npad_for(n)
  rpt = npad // _NS
  nef = nch // _NC

  @functools.partial(
      pl.kernel,
      out_type=[
          jax.ShapeDtypeStruct((_NC, npad, dh), jnp.float32),
          jax.ShapeDtypeStruct((_NC, npad, de), jnp.float32),
      ],
      mesh=_mesh(),
      scratch_types=[
          pltpu.VMEM((ept,), jnp.int32),
          pltpu.VMEM((ept,), jnp.int32),
          pltpu.VMEM((_C, dh), jnp.float32),
          pltpu.VMEM((_C, dh), jnp.float32),
          pltpu.VMEM((_C, dh), jnp.float32),
          pltpu.VMEM((_C, dh), jnp.float32),
          pltpu.VMEM((_C, dh), jnp.float32),
          pltpu.VMEM((_C, de), jnp.float32),
          pltpu.VMEM_SHARED((npad, dh), jnp.float32),
          pltpu.VMEM_SHARED((npad, de), jnp.float32),
          pltpu.SemaphoreType.DMA,
          pltpu.SemaphoreType.DMA,
          pltpu.SemaphoreType.DMA,
          pltpu.SemaphoreType.DMA,
      ],
      compiler_params=_LINEAR,
  )
  def k(eidx_h, feat2_h, psum_h, pdeg_h,
        sidx, didx, r0b, r1b, r2b, r3b, zb, onesb, acc, dacc,
        g0, g1, g2, g3):
    cid = lax.axis_index("c")
    sid = lax.axis_index("s")
    base = sid * ept

    pltpu.sync_copy(eidx_h.at[cid, pl.ds(base, ept)], sidx)
    pltpu.sync_copy(eidx_h.at[2, pl.ds(base, ept)], didx)

    def gsrc(ci):
      return feat2_h.at[sidx.at[pl.ds(ci * _C, _C)]]

    def dst_at(ci):
      return didx.at[pl.ds(ci * _C, _C)]

    ring = ((r0b, g0), (r1b, g1), (r2b, g2), (r3b, g3))

    pltpu.async_copy(gsrc(0), r0b, g0)
    pltpu.async_copy(gsrc(1), r1b, g1)

    zeros16 = jnp.zeros((16,), jnp.float32)
    ones16 = jnp.ones((16,), jnp.float32)

    def zrow(i, carry):
      for j in range(dh // 16):
        zb[i, pl.ds(j * 16, 16)] = zeros16
      for j in range(de // 16):
        onesb[i, pl.ds(j * 16, 16)] = ones16
      return carry
    lax.fori_loop(0, _C, zrow, 0)

    rr0 = sid * rpt
    for kk in range(rpt // _C):
      pltpu.sync_copy(zb, acc.at[pl.ds(rr0 + kk * _C, _C)])
      pltpu.sync_copy(zb.at[:, pl.ds(0, de)],
                      dacc.at[pl.ds(rr0 + kk * _C, _C)])
    plsc.subcore_barrier()

    def gwait(buf, sem, ci):
      pltpu.make_async_copy(gsrc(ci), buf, sem).wait()

    def sissue(buf, sem, ci):
      pltpu.make_async_copy(buf, acc.at[dst_at(ci)], sem).start(add=True)

    def swait(buf, sem, ci):
      pltpu.make_async_copy(buf, acc.at[dst_at(ci)], sem).wait()

    eoff = cid * nef

    gwait(r0b, g0, 0)
    sissue(r0b, g0, 0)
    pltpu.async_copy(gsrc(2), r2b, g2)
    gwait(r1b, g1, 1)
    sissue(r1b, g1, 1)
    pltpu.async_copy(gsrc(3), r3b, g3)

    def body(g, carry):
      for off in range(4):
        ci = 4 * g + 2 + off
        buf, sem = ring[(2 + off) % 4]
        gwait(buf, sem, ci)
        sissue(buf, sem, ci)
        nbuf, nsem = ring[(4 + off) % 4]
        swait(nbuf, nsem, ci - 2)

        @pl.when(ci + 2 < nch)
        def _():
          pltpu.async_copy(gsrc(ci + 2), nbuf, nsem)
      pltpu.sync_copy(onesb, dacc.at[dst_at(eoff + 2 * g)], add=True)
      pltpu.sync_copy(onesb, dacc.at[dst_at(eoff + 2 * g + 1)], add=True)
      return carry

    niter = (nch - 2) // 4
    lax.fori_loop(0, niter, body, 0)

    swait(ring[0][0], ring[0][1], nch - 2)
    swait(ring[1][0], ring[1][1], nch - 1)
    for ci in range(2 * niter, nef):
      pltpu.sync_copy(onesb, dacc.at[dst_at(eoff + ci)], add=True)

    plsc.subcore_barrier()
    pltpu.sync_copy(acc.at[pl.ds(rr0, rpt)],
                    psum_h.at[cid, pl.ds(rr0, rpt)])
    pltpu.sync_copy(dacc.at[pl.ds(rr0, rpt)],
                    pdeg_h.at[cid, pl.ds(rr0, rpt)])

  return k(eidx3, feat2)


def _sc_ef_sums(ef, eidx3, n):
  e_total = eidx3.shape[-1]
  de = ef.shape[-1]
  ept = e_total // _NS
  nch = ept // _C
  npad = _npad_for(n)
  rpt = npad // _NS
  nef = nch // _NC

  @functools.partial(
      pl.kernel,
      out_type=jax.ShapeDtypeStruct((_NC, npad, de), jnp.float32),
      mesh=_mesh(),
      scratch_types=[
          pltpu.VMEM((ept,), jnp.int32),
          pltpu.VMEM((_C, de), jnp.float32),
          pltpu.VMEM((_C, de), jnp.float32),
          pltpu.VMEM((_C, de), jnp.float32),
          pltpu.VMEM((_C, de), jnp.float32),
          pltpu.VMEM((_C, de), jnp.float32),
          pltpu.VMEM_SHARED((npad, de), jnp.float32),
          pltpu.SemaphoreType.DMA,
          pltpu.SemaphoreType.DMA,
          pltpu.SemaphoreType.DMA,
          pltpu.SemaphoreType.DMA,
      ],
      compiler_params=_LINEAR,
  )
  def k(eidx_h, ef_h, pef_h, didx, e0b, e1b, e2b, e3b, zb, eacc,
        e0, e1, e2, e3):
    cid = lax.axis_index("c")
    sid = lax.axis_index("s")
    base = sid * ept
    eoff = cid * nef

    pltpu.sync_copy(eidx_h.at[2, pl.ds(base, ept)], didx)

    def ef_at(ei):
      return ef_h.at[pl.ds(base + ei * _C, _C)]

    def dst_at(ci):
      return didx.at[pl.ds(ci * _C, _C)]

    ring = ((e0b, e0), (e1b, e1), (e2b, e2), (e3b, e3))

    pltpu.async_copy(ef_at(eoff), e0b, e0)
    pltpu.async_copy(ef_at(eoff + 1), e1b, e1)

    zeros16 = jnp.zeros((16,), jnp.float32)

    def zrow(i, carry):
      for j in range(de // 16):
        zb[i, pl.ds(j * 16, 16)] = zeros16
      return carry
    lax.fori_loop(0, _C, zrow, 0)

    rr0 = sid * rpt
    for kk in range(rpt // _C):
      pltpu.sync_copy(zb, eacc.at[pl.ds(rr0 + kk * _C, _C)])
    plsc.subcore_barrier()

    def ewait(buf, sem, lc):
      pltpu.make_async_copy(ef_at(eoff + lc), buf, sem).wait()

    def sissue(buf, sem, lc):
      pltpu.make_async_copy(buf, eacc.at[dst_at(eoff + lc)],
                            sem).start(add=True)

    def swait(buf, sem, lc):
      pltpu.make_async_copy(buf, eacc.at[dst_at(eoff + lc)], sem).wait()

    ewait(e0b, e0, 0)
    sissue(e0b, e0, 0)
    pltpu.async_copy(ef_at(eoff + 2), e2b, e2)
    ewait(e1b, e1, 1)
    sissue(e1b, e1, 1)
    pltpu.async_copy(ef_at(eoff + 3), e3b, e3)

    def body(g, carry):
      for off in range(4):
        lc = 4 * g + 2 + off

        @pl.when(lc < nef)
        def _():
          buf, sem = ring[(2 + off) % 4]
          ewait(buf, sem, lc)
          sissue(buf, sem, lc)
          nbuf, nsem = ring[off % 4]
          swait(nbuf, nsem, lc - 2)

          @pl.when(lc + 2 < nef)
          def _():
            pltpu.async_copy(ef_at(eoff + lc + 2), nbuf, nsem)
      return carry

    lax.fori_loop(0, (nef + 3) // 4, body, 0)

    lc1, lc2 = nef - 2, nef - 1
    b1, s1 = ring[(2 + (lc1 - 2) % 4) % 4]
    b2, s2 = ring[(2 + (lc2 - 2) % 4) % 4]
    swait(b1, s1, lc1)
    swait(b2, s2, lc2)

    plsc.subcore_barrier()
    pltpu.sync_copy(eacc.at[pl.ds(rr0, rpt)],
                    pef_h.at[cid, pl.ds(rr0, rpt)])

  return k(eidx3, ef)


def _self_term(feat, w_self, bias2d):
  n, d = feat.shape
  blk = 1000
  grid = (n // blk,)

  def body(f_ref, ws_ref, b_ref, o_ref):
    o_ref[...] = (jnp.dot(f_ref[...], ws_ref[...],
                          preferred_element_type=jnp.float32) + b_ref[...])

  return pl.pallas_call(
      body,
      grid=grid,
      in_specs=[
          pl.BlockSpec((blk, d), lambda i: (i, 0)),
          pl.BlockSpec((d, d), lambda i: (0, 0)),
          pl.BlockSpec((1, d), lambda i: (0, 0)),
      ],
      out_specs=pl.BlockSpec((blk, d), lambda i: (i, 0)),
      out_shape=jax.ShapeDtypeStruct((n, d), jnp.float32),
  )(feat, w_self, bias2d)


def _combine(selfterm, psum, pdeg, pef, w_neigh):
  n, d = selfterm.shape
  dh = psum.shape[-1]
  de = pef.shape[-1]
  dk = w_neigh.shape[0]
  blk = 1000
  grid = (n // blk,)

  def body(st_ref, ps_ref, pd_ref, pe_ref, wn_ref, o_ref):
    dg = pd_ref[0, :, 0:1] + pd_ref[1, :, 0:1]
    e = pe_ref[0] + pe_ref[1]
    scale = 1.0 / jnp.maximum(dg, 1.0)
    wn = wn_ref[...]
    h = (jnp.dot(ps_ref[0], wn[0:dh], preferred_element_type=jnp.float32)
         + jnp.dot(ps_ref[1], wn[dh:2 * dh],
                   preferred_element_type=jnp.float32)
         + jnp.dot(e, wn[2 * dh:dk], preferred_element_type=jnp.float32))
    o_ref[...] = st_ref[...] + h * scale

  return pl.pallas_call(
      body,
      grid=grid,
      in_specs=[
          pl.BlockSpec((blk, d), lambda i: (i, 0)),
          pl.BlockSpec((_NC, blk, dh), lambda i: (0, i, 0)),
          pl.BlockSpec((_NC, blk, de), lambda i: (0, i, 0)),
          pl.BlockSpec((_NC, blk, de), lambda i: (0, i, 0)),
          pl.BlockSpec((dk, d), lambda i: (0, 0)),
      ],
      out_specs=pl.BlockSpec((blk, d), lambda i: (i, 0)),
      out_shape=jax.ShapeDtypeStruct((n, d), jnp.float32),
  )(selfterm, psum, pdeg, pef, w_neigh)


def kernel(feat, edge_index, edge_feats, W_self, W_neigh, bias):
  n, d = feat.shape
  de = edge_feats.shape[1]
  dh = d // _NC
  src = edge_index[0]
  dst = edge_index[1]
  feat2 = jnp.concatenate([feat[:, :dh], feat[:, dh:]], axis=0)
  eidx3 = jnp.stack([src, src + n, dst])
  psum, pdeg = _sc_feat_deg_sums(feat2, eidx3, n, de)
  pef = _sc_ef_sums(edge_feats, eidx3, n)
  st = _self_term(feat, W_self, bias.reshape(1, d))
  return _combine(st, psum, pdeg, pef, W_neigh)
